# baseline (speedup 1.0000x reference)
import functools

import jax
import jax.numpy as jnp
from jax import lax
from jax.experimental import pallas as pl
from jax.experimental.pallas import tpu as pltpu
from jax.experimental.pallas import tpu_sc as plsc

N = 10000
E = 320000
EMBED = 768
HID = 128
HEADS = 4
HEAD_DIM = 32
PE_DIM = 192
SROW = 256
DROW = 128

BN = 400
GRID = N // BN

NTILES = 32
EPT = E // NTILES
K = 40
NCHUNK = EPT // K
NMSG = 10240
NDEN = 320
NACC = 10752
RPT = NACC // 16
ZR = 24


def _leaky(x, slope):
    return jnp.maximum(x, slope * x)


def _frontend_body(cell_ref, ent_ref, pos_ref, eW_ref, eb_ref, inW_ref, inb_ref,
                   g_ref, bta_ref, W1_ref, b1_ref, As_ref, Ad_ref,
                   h0_ref, S_ref, D_ref, mx_ref):
    i = pl.program_id(0)
    cell = cell_ref[...]
    ent = ent_ref[...]
    ent_proj = jnp.dot(ent, eW_ref[...], preferred_element_type=jnp.float32) + eb_ref[...]

    lane = lax.broadcasted_iota(jnp.int32, (1, PE_DIM), 1)
    i2 = (lane // 2) * 2
    div2 = jnp.exp(i2.astype(jnp.float32) * (-jnp.log(10000.0) / PE_DIM))
    even = (lane % 2) == 0

    def pe(pos_col):
        ang = pos_col * div2
        return jnp.where(even, jnp.sin(ang), jnp.cos(ang))

    per = pe(pos_ref[:, 0:1])
    pec = pe(pos_ref[:, 1:2])

    h = (jnp.dot(cell, inW_ref[0:EMBED, :], preferred_element_type=jnp.float32)
         + jnp.dot(per, inW_ref[EMBED:EMBED + PE_DIM, :], preferred_element_type=jnp.float32)
         + jnp.dot(pec, inW_ref[EMBED + PE_DIM:EMBED + 2 * PE_DIM, :], preferred_element_type=jnp.float32)
         + jnp.dot(ent_proj, inW_ref[EMBED + 2 * PE_DIM:, :], preferred_element_type=jnp.float32)
         + inb_ref[...])
    mu = jnp.mean(h, axis=-1, keepdims=True)
    var = jnp.mean((h - mu) ** 2, axis=-1, keepdims=True)
    h = (h - mu) / jnp.sqrt(var + 1e-5) * g_ref[...] + bta_ref[...]
    h = jnp.maximum(h, 0.0)
    h0_ref[...] = h

    hp = jnp.dot(h, W1_ref[...], preferred_element_type=jnp.float32) + b1_ref[...]
    sa = jnp.dot(hp, As_ref[...], preferred_element_type=jnp.float32)
    S_ref[:, 0:HID] = hp
    S_ref[:, HID:HID + 16] = sa
    D_ref[:, 0:16] = jnp.dot(hp, Ad_ref[...], preferred_element_type=jnp.float32)

    m = jnp.broadcast_to(jnp.max(sa, axis=0, keepdims=True), (8, 16))

    @pl.when(i == 0)
    def _():
        mx_ref[...] = m

    @pl.when(i > 0)
    def _():
        mx_ref[...] = jnp.maximum(mx_ref[...], m)


def _frontend(cell, ent, pos, eW, eb, inW, inb, g, bta, W1, b1, As, Ad):
    full = lambda shape: pl.BlockSpec(shape, lambda i: tuple(0 for _ in shape))
    blk = lambda shape: pl.BlockSpec(shape, lambda i: (i,) + tuple(0 for _ in shape[1:]))
    return pl.pallas_call(
        _frontend_body,
        grid=(GRID,),
        in_specs=[blk((BN, EMBED)), blk((BN, 256)), blk((BN, 2)),
                  full((256, EMBED)), full((1, EMBED)), full((1920, HID)),
                  full((1, HID)), full((1, HID)), full((1, HID)),
                  full((HID, HID)), full((1, HID)), full((HID, 16)), full((HID, 16))],
        out_specs=[blk((BN, HID)), blk((BN, SROW)), blk((BN, DROW)), full((8, 16))],
        out_shape=[jax.ShapeDtypeStruct((N, HID), jnp.float32),
                   jax.ShapeDtypeStruct((N, SROW), jnp.float32),
                   jax.ShapeDtypeStruct((N, DROW), jnp.float32),
                   jax.ShapeDtypeStruct((8, 16), jnp.float32)],
    )(cell, ent, pos, eW, eb, inW, inb, g, bta, W1, b1, As, Ad)


def _sel_matrix():
    r = lax.broadcasted_iota(jnp.int32, (8, HID), 0)
    c = lax.broadcasted_iota(jnp.int32, (8, HID), 1)
    return ((c // HEAD_DIM) == r).astype(jnp.float32)


def _merge(m0_ref, m1_ref, d0_ref, d1_ref, hprev_ref):
    msg = m0_ref[...] + m1_ref[...]
    d8 = d0_ref[...] + d1_ref[...]
    dfull = jnp.dot(d8, _sel_matrix(), preferred_element_type=jnp.float32)
    safe = jnp.where(dfull > 0.0, dfull, 1.0)
    return _leaky(msg / safe + hprev_ref[...], 0.01)


def _combine_prep_body(m0_ref, m1_ref, d0_ref, d1_ref, hprev_ref,
                       W_ref, b_ref, As_ref, Ad_ref,
                       h1_ref, S_ref, D_ref, mx_ref):
    i = pl.program_id(0)
    h = _merge(m0_ref, m1_ref, d0_ref, d1_ref, hprev_ref)
    h1_ref[...] = h
    hp = jnp.dot(h, W_ref[...], preferred_element_type=jnp.float32) + b_ref[...]
    sa = jnp.dot(hp, As_ref[...], preferred_element_type=jnp.float32)
    S_ref[:, 0:HID] = hp
    S_ref[:, HID:HID + 16] = sa
    D_ref[:, 0:16] = jnp.dot(hp, Ad_ref[...], preferred_element_type=jnp.float32)
    m = jnp.broadcast_to(jnp.max(sa, axis=0, keepdims=True), (8, 16))

    @pl.when(i == 0)
    def _():
        mx_ref[...] = m

    @pl.when(i > 0)
    def _():
        mx_ref[...] = jnp.maximum(mx_ref[...], m)


def _combine_prep(m0, m1, d0, d1, hprev, W, b, As, Ad):
    full = lambda shape: pl.BlockSpec(shape, lambda i: tuple(0 for _ in shape))
    blk = lambda shape: pl.BlockSpec(shape, lambda i: (i,) + tuple(0 for _ in shape[1:]))
    return pl.pallas_call(
        _combine_prep_body,
        grid=(GRID,),
        in_specs=[blk((BN, HID)), blk((BN, HID)), blk((BN, 8)), blk((BN, 8)),
                  blk((BN, HID)),
                  full((HID, HID)), full((1, HID)), full((HID, 16)), full((HID, 16))],
        out_specs=[blk((BN, HID)), blk((BN, SROW)), blk((BN, DROW)), full((8, 16))],
        out_shape=[jax.ShapeDtypeStruct((N, HID), jnp.float32),
                   jax.ShapeDtypeStruct((N, SROW), jnp.float32),
                   jax.ShapeDtypeStruct((N, DROW), jnp.float32),
                   jax.ShapeDtypeStruct((8, 16), jnp.float32)],
    )(m0, m1, d0, d1, hprev, W, b, As, Ad)


def _final_body(m0_ref, m1_ref, d0_ref, d1_ref, hprev_ref, out_ref):
    out_ref[...] = _merge(m0_ref, m1_ref, d0_ref, d1_ref, hprev_ref)


def _final(m0, m1, d0, d1, hprev):
    blk = lambda shape: pl.BlockSpec(shape, lambda i: (i,) + tuple(0 for _ in shape[1:]))
    return pl.pallas_call(
        _final_body,
        grid=(GRID,),
        in_specs=[blk((BN, HID)), blk((BN, HID)), blk((BN, 8)), blk((BN, 8)),
                  blk((BN, HID))],
        out_specs=blk((BN, HID)),
        out_shape=jax.ShapeDtypeStruct((N, HID), jnp.float32),
    )(m0, m1, d0, d1, hprev)


def _lane_bcast(v, h):
    return jnp.take_along_axis(v, jnp.full((16,), h, dtype=jnp.int32), axis=0)


def _sc_body(S_hbm, D_hbm, mx_hbm, src_hbm, dst_hbm, out_hbm,
             srcv0, srcv1, dstv0, dstv1, dstv20, dstv21, srows0, srows1,
             drows0, drows1, mrows20, mrows21, maxv, zbuf, acc,
             semS0, semS1, semD0, semD1, semW0, semW1, semW20, semW21):
    cid = lax.axis_index("c")
    sid = lax.axis_index("s")
    wid = cid * 16 + sid
    iota16 = lax.broadcasted_iota(jnp.int32, (16,), 0)
    zeros16 = jnp.zeros((16,), jnp.float32)
    dmask = iota16 < HEADS
    srcv = (srcv0, srcv1)
    dstv = (dstv0, dstv1)
    dstv2 = (dstv20, dstv21)
    srows = (srows0, srows1)
    drows = (drows0, drows1)
    mrows2 = (mrows20, mrows21)
    semS = (semS0, semS1)
    semD = (semD0, semD1)
    semW = (semW0, semW1)
    semW2 = (semW20, semW21)

    def zrow(j, carry):
        for v in range(8):
            zbuf[j, pl.ds(v * 16, 16)] = zeros16
        return carry
    lax.fori_loop(0, ZR, zrow, 0)
    for r in range(RPT // ZR):
        pltpu.sync_copy(zbuf, acc.at[pl.ds(sid * RPT + r * ZR, ZR)])
    plsc.subcore_barrier()

    pltpu.sync_copy(mx_hbm, maxv)
    mv = maxv[...]
    ebase = wid * EPT

    def issue(j, b, first=False):
        if not first:
            @pl.when(j >= 2)
            def _():
                pltpu.make_async_copy(drows[b], acc.at[dstv[b]], semW[b]).wait()
        off = ebase + j * K
        pltpu.sync_copy(src_hbm.at[pl.ds(off, K)], srcv[b])
        pltpu.sync_copy(dst_hbm.at[pl.ds(off, K)], dstv[b])
        pltpu.async_copy(S_hbm.at[srcv[b]], srows[b], semS[b])
        pltpu.async_copy(D_hbm.at[dstv[b]], drows[b], semD[b])

    def run_chunk(j, b):
        sr = srows[b]
        dr = drows[b]
        dv_ref = dstv[b]
        m2 = mrows2[b]

        @pl.when(j >= 2)
        def _():
            pltpu.make_async_copy(m2, acc.at[dstv2[b]], semW2[b]).wait()
        for r in range((K + 15) // 16):
            bb = min(16 * r, K - 16)
            dv = dv_ref[pl.ds(bb, 16)]
            dstv2[b][pl.ds(bb, 16)] = (dv >> 5) + NMSG

        @plsc.parallel_loop(0, K, step=1, unroll=2)
        def zden(i):
            for v in range(8):
                m2[i, pl.ds(v * 16, 16)] = zeros16

        @plsc.parallel_loop(0, K, step=1, unroll=4)
        def edge(i):
            sa = sr[i, pl.ds(HID, 16)]
            dc = dr[i, pl.ds(0, 16)]
            x = sa + dc
            e = jnp.maximum(x, 0.2 * x)
            y = mv + dc
            cc = jnp.maximum(y, 0.2 * y)
            ex = jnp.exp(e - cc)
            base = jnp.minimum((i // 16) * 16, K - 16)
            dvec = dv_ref[pl.ds(base, 16)]
            d_b = jnp.take_along_axis(dvec, jnp.full((16,), i - base, jnp.int32),
                                      axis=0)
            plsc.store_scatter(m2, [jnp.full((16,), i, jnp.int32),
                                    (d_b & 31) * 4 + iota16], ex, mask=dmask)
            for h in range(HEADS):
                bh = _lane_bcast(ex, h)
                dr[i, pl.ds(h * 32, 16)] = sr[i, pl.ds(h * 32, 16)] * bh
                dr[i, pl.ds(h * 32 + 16, 16)] = sr[i, pl.ds(h * 32 + 16, 16)] * bh
        pltpu.async_copy(dr, acc.at[dv_ref], semW[b], add=True)
        pltpu.async_copy(m2, acc.at[dstv2[b]], semW2[b], add=True)

    issue(0, 0, first=True)

    def outer(g, carry):
        for b in range(2):
            j = 2 * g + b
            pltpu.make_async_copy(S_hbm.at[srcv[b]], srows[b], semS[b]).wait()
            pltpu.make_async_copy(D_hbm.at[dstv[b]], drows[b], semD[b]).wait()
            if b == 0:
                issue(j + 1, 1)
            else:
                @pl.when(g < NCHUNK // 2 - 1)
                def _():
                    issue(j + 1, 0)
            run_chunk(j, b)
        return carry
    lax.fori_loop(0, NCHUNK // 2, outer, 0)

    for b in range(2):
        pltpu.make_async_copy(drows[b], acc.at[dstv[b]], semW[b]).wait()
        pltpu.make_async_copy(mrows2[b], acc.at[dstv2[b]], semW2[b]).wait()

    plsc.subcore_barrier()
    for r in range(RPT // ZR):
        start = sid * RPT + r * ZR
        pltpu.sync_copy(acc.at[pl.ds(start, ZR)], out_hbm.at[cid, pl.ds(start, ZR)])


@functools.lru_cache(maxsize=1)
def _sc_edge_fn():
    return pl.kernel(
        _sc_body,
        out_type=jax.ShapeDtypeStruct((2, NACC, HID), jnp.float32),
        mesh=plsc.VectorSubcoreMesh(core_axis_name="c", subcore_axis_name="s"),
        compiler_params=pltpu.CompilerParams(needs_layout_passes=False),
        scratch_types=[
            pltpu.VMEM((K,), jnp.int32),
            pltpu.VMEM((K,), jnp.int32),
            pltpu.VMEM((K,), jnp.int32),
            pltpu.VMEM((K,), jnp.int32),
            pltpu.VMEM((K,), jnp.int32),
            pltpu.VMEM((K,), jnp.int32),
            pltpu.VMEM((K, SROW), jnp.float32),
            pltpu.VMEM((K, SROW), jnp.float32),
            pltpu.VMEM((K, DROW), jnp.float32),
            pltpu.VMEM((K, DROW), jnp.float32),
            pltpu.VMEM((K, HID), jnp.float32),
            pltpu.VMEM((K, HID), jnp.float32),
            pltpu.VMEM((16,), jnp.float32),
            pltpu.VMEM((ZR, HID), jnp.float32),
            pltpu.VMEM_SHARED((NACC, HID), jnp.float32),
            pltpu.SemaphoreType.DMA,
            pltpu.SemaphoreType.DMA,
            pltpu.SemaphoreType.DMA,
            pltpu.SemaphoreType.DMA,
            pltpu.SemaphoreType.DMA,
            pltpu.SemaphoreType.DMA,
            pltpu.SemaphoreType.DMA,
            pltpu.SemaphoreType.DMA,
        ],
    )


def _sc_edge(S, D, mxv, src, dst):
    return _sc_edge_fn()(S, D, mxv, src, dst)


def _mix_mat(a):
    return (a[:, :, None] * jnp.eye(HEADS, 16, dtype=jnp.float32)[:, None, :]).reshape(HID, 16)


def _mx_vec(mx):
    return jnp.concatenate([mx[0, :HEADS], jnp.full((16 - HEADS,), 1e9, jnp.float32)])


def _split_acc(acc):
    msg = acc[:, :N, :]
    nden_rows = (N + 31) // 32
    den = acc[:, NMSG:NMSG + nden_rows, :].reshape(2, nden_rows * HID)
    den = den[:, :N * HEADS].reshape(2, N, HEADS)
    den = jnp.concatenate([den, jnp.zeros((2, N, 4), jnp.float32)], axis=-1)
    return msg, den


def kernel(external_cell_embeds, entity_embeddings, edge_index, row_indices, col_indices,
           entity_W, entity_b, in_W, in_b, ln_g, ln_b,
           W1, b1, as1, ad1, W2, b2, as2, ad2):
    edge = edge_index.astype(jnp.int32)
    src = edge[0]
    dst = edge[1]
    pos = jnp.stack([row_indices.astype(jnp.float32),
                     col_indices.astype(jnp.float32)], axis=-1)

    h0, S1, D1, mx1 = _frontend(
        external_cell_embeds, entity_embeddings, pos,
        entity_W, entity_b.reshape(1, -1), in_W, in_b.reshape(1, -1),
        ln_g.reshape(1, -1), ln_b.reshape(1, -1),
        W1, b1.reshape(1, -1), _mix_mat(as1), _mix_mat(ad1))

    b2r = b2.reshape(1, -1)
    As2, Ad2 = _mix_mat(as2), _mix_mat(ad2)

    nlayers = 2 + jnp.minimum(src[0], 0) * 0

    def cond(carry):
        return carry[0] < nlayers

    def body(carry):
        t, S, D, mxv, h = carry
        acc = _sc_edge(S, D, mxv, src, dst)
        msg, den = _split_acc(acc)
        hn, Sn, Dn, mxn = _combine_prep(msg[0], msg[1], den[0], den[1], h,
                                        W2, b2r, As2, Ad2)
        return (t + 1, Sn, Dn, _mx_vec(mxn), hn)

    carry = lax.while_loop(cond, body, (jnp.int32(0), S1, D1, _mx_vec(mx1), h0))
    return carry[4]


# speedup vs baseline: 1.0135x; 1.0135x over previous
import functools

import jax
import jax.numpy as jnp
from jax import lax
from jax.experimental import pallas as pl
from jax.experimental.pallas import tpu as pltpu
from jax.experimental.pallas import tpu_sc as plsc

N = 10000
E = 320000
EMBED = 768
HID = 128
HEADS = 4
HEAD_DIM = 32
PE_DIM = 192
SROW = 256
DROW = 128

BN = 400
GRID = N // BN

NTILES = 32
EPT = E // NTILES
K = 40
NCHUNK = EPT // K
NMSG = 10240
NDEN = 320
NACC = 10752
RPT = NACC // 16
ZR = 24


def _leaky(x, slope):
    return jnp.maximum(x, slope * x)


def _frontend_body(cell_ref, ent_ref, pos_ref, eW_ref, eb_ref, inW_ref, inb_ref,
                   g_ref, bta_ref, W1_ref, b1_ref, As_ref, Ad_ref,
                   h0_ref, S_ref, D_ref, mx_ref):
    i = pl.program_id(0)
    cell = cell_ref[...]
    ent = ent_ref[...]
    ent_proj = jnp.dot(ent, eW_ref[...], preferred_element_type=jnp.float32) + eb_ref[...]

    lane = lax.broadcasted_iota(jnp.int32, (1, PE_DIM), 1)
    i2 = (lane // 2) * 2
    div2 = jnp.exp(i2.astype(jnp.float32) * (-jnp.log(10000.0) / PE_DIM))
    even = (lane % 2) == 0

    def pe(pos_col):
        ang = pos_col * div2
        return jnp.where(even, jnp.sin(ang), jnp.cos(ang))

    per = pe(pos_ref[:, 0:1])
    pec = pe(pos_ref[:, 1:2])

    h = (jnp.dot(cell, inW_ref[0:EMBED, :], preferred_element_type=jnp.float32)
         + jnp.dot(per, inW_ref[EMBED:EMBED + PE_DIM, :], preferred_element_type=jnp.float32)
         + jnp.dot(pec, inW_ref[EMBED + PE_DIM:EMBED + 2 * PE_DIM, :], preferred_element_type=jnp.float32)
         + jnp.dot(ent_proj, inW_ref[EMBED + 2 * PE_DIM:, :], preferred_element_type=jnp.float32)
         + inb_ref[...])
    mu = jnp.mean(h, axis=-1, keepdims=True)
    var = jnp.mean((h - mu) ** 2, axis=-1, keepdims=True)
    h = (h - mu) / jnp.sqrt(var + 1e-5) * g_ref[...] + bta_ref[...]
    h = jnp.maximum(h, 0.0)
    h0_ref[...] = h

    hp = jnp.dot(h, W1_ref[...], preferred_element_type=jnp.float32) + b1_ref[...]
    sa = jnp.dot(hp, As_ref[...], preferred_element_type=jnp.float32)
    S_ref[:, 0:HID] = hp
    S_ref[:, HID:HID + 16] = sa
    D_ref[:, 0:16] = jnp.dot(hp, Ad_ref[...], preferred_element_type=jnp.float32)

    m = jnp.broadcast_to(jnp.max(sa, axis=0, keepdims=True), (8, 16))

    @pl.when(i == 0)
    def _():
        mx_ref[...] = m

    @pl.when(i > 0)
    def _():
        mx_ref[...] = jnp.maximum(mx_ref[...], m)


def _frontend(cell, ent, pos, eW, eb, inW, inb, g, bta, W1, b1, As, Ad):
    full = lambda shape: pl.BlockSpec(shape, lambda i: tuple(0 for _ in shape))
    blk = lambda shape: pl.BlockSpec(shape, lambda i: (i,) + tuple(0 for _ in shape[1:]))
    return pl.pallas_call(
        _frontend_body,
        grid=(GRID,),
        in_specs=[blk((BN, EMBED)), blk((BN, 256)), blk((BN, 2)),
                  full((256, EMBED)), full((1, EMBED)), full((1920, HID)),
                  full((1, HID)), full((1, HID)), full((1, HID)),
                  full((HID, HID)), full((1, HID)), full((HID, 16)), full((HID, 16))],
        out_specs=[blk((BN, HID)), blk((BN, SROW)), blk((BN, DROW)), full((8, 16))],
        out_shape=[jax.ShapeDtypeStruct((N, HID), jnp.float32),
                   jax.ShapeDtypeStruct((N, SROW), jnp.float32),
                   jax.ShapeDtypeStruct((N, DROW), jnp.float32),
                   jax.ShapeDtypeStruct((8, 16), jnp.float32)],
    )(cell, ent, pos, eW, eb, inW, inb, g, bta, W1, b1, As, Ad)


def _sel_matrix():
    r = lax.broadcasted_iota(jnp.int32, (8, HID), 0)
    c = lax.broadcasted_iota(jnp.int32, (8, HID), 1)
    return ((c // HEAD_DIM) == r).astype(jnp.float32)


def _merge(m0_ref, m1_ref, d0_ref, d1_ref, hprev_ref):
    msg = m0_ref[...] + m1_ref[...]
    d8 = d0_ref[...] + d1_ref[...]
    dfull = jnp.dot(d8, _sel_matrix(), preferred_element_type=jnp.float32)
    safe = jnp.where(dfull > 0.0, dfull, 1.0)
    return _leaky(msg / safe + hprev_ref[...], 0.01)


def _combine_prep_body(m0_ref, m1_ref, d0_ref, d1_ref, hprev_ref,
                       W_ref, b_ref, As_ref, Ad_ref,
                       h1_ref, S_ref, D_ref, mx_ref):
    i = pl.program_id(0)
    h = _merge(m0_ref, m1_ref, d0_ref, d1_ref, hprev_ref)
    h1_ref[...] = h
    hp = jnp.dot(h, W_ref[...], preferred_element_type=jnp.float32) + b_ref[...]
    sa = jnp.dot(hp, As_ref[...], preferred_element_type=jnp.float32)
    S_ref[:, 0:HID] = hp
    S_ref[:, HID:HID + 16] = sa
    D_ref[:, 0:16] = jnp.dot(hp, Ad_ref[...], preferred_element_type=jnp.float32)
    m = jnp.broadcast_to(jnp.max(sa, axis=0, keepdims=True), (8, 16))

    @pl.when(i == 0)
    def _():
        mx_ref[...] = m

    @pl.when(i > 0)
    def _():
        mx_ref[...] = jnp.maximum(mx_ref[...], m)


def _combine_prep(m0, m1, d0, d1, hprev, W, b, As, Ad):
    full = lambda shape: pl.BlockSpec(shape, lambda i: tuple(0 for _ in shape))
    blk = lambda shape: pl.BlockSpec(shape, lambda i: (i,) + tuple(0 for _ in shape[1:]))
    return pl.pallas_call(
        _combine_prep_body,
        grid=(GRID,),
        in_specs=[blk((BN, HID)), blk((BN, HID)), blk((BN, 8)), blk((BN, 8)),
                  blk((BN, HID)),
                  full((HID, HID)), full((1, HID)), full((HID, 16)), full((HID, 16))],
        out_specs=[blk((BN, HID)), blk((BN, SROW)), blk((BN, DROW)), full((8, 16))],
        out_shape=[jax.ShapeDtypeStruct((N, HID), jnp.float32),
                   jax.ShapeDtypeStruct((N, SROW), jnp.float32),
                   jax.ShapeDtypeStruct((N, DROW), jnp.float32),
                   jax.ShapeDtypeStruct((8, 16), jnp.float32)],
    )(m0, m1, d0, d1, hprev, W, b, As, Ad)


def _final_body(m0_ref, m1_ref, d0_ref, d1_ref, hprev_ref, out_ref):
    out_ref[...] = _merge(m0_ref, m1_ref, d0_ref, d1_ref, hprev_ref)


def _final(m0, m1, d0, d1, hprev):
    blk = lambda shape: pl.BlockSpec(shape, lambda i: (i,) + tuple(0 for _ in shape[1:]))
    return pl.pallas_call(
        _final_body,
        grid=(GRID,),
        in_specs=[blk((BN, HID)), blk((BN, HID)), blk((BN, 8)), blk((BN, 8)),
                  blk((BN, HID))],
        out_specs=blk((BN, HID)),
        out_shape=jax.ShapeDtypeStruct((N, HID), jnp.float32),
    )(m0, m1, d0, d1, hprev)


def _lane_bcast(v, h):
    return jnp.take_along_axis(v, jnp.full((16,), h, dtype=jnp.int32), axis=0)


def _sc_body(S_hbm, D_hbm, mx_hbm, src_hbm, dst_hbm, out_hbm,
             srcv0, srcv1, dstv0, dstv1, dstv20, dstv21, srows0, srows1,
             drows0, drows1, mrows20, mrows21, maxv, zbuf, acc,
             semS0, semS1, semD0, semD1, semW0, semW1, semW20, semW21):
    cid = lax.axis_index("c")
    sid = lax.axis_index("s")
    wid = cid * 16 + sid
    iota16 = lax.broadcasted_iota(jnp.int32, (16,), 0)
    zeros16 = jnp.zeros((16,), jnp.float32)
    dmask = iota16 < HEADS
    srcv = (srcv0, srcv1)
    dstv = (dstv0, dstv1)
    dstv2 = (dstv20, dstv21)
    srows = (srows0, srows1)
    drows = (drows0, drows1)
    mrows2 = (mrows20, mrows21)
    semS = (semS0, semS1)
    semD = (semD0, semD1)
    semW = (semW0, semW1)
    semW2 = (semW20, semW21)

    def zrow(j, carry):
        for v in range(8):
            zbuf[j, pl.ds(v * 16, 16)] = zeros16
        return carry
    lax.fori_loop(0, ZR, zrow, 0)
    for r in range(RPT // ZR):
        pltpu.sync_copy(zbuf, acc.at[pl.ds(sid * RPT + r * ZR, ZR)])
    plsc.subcore_barrier()

    pltpu.sync_copy(mx_hbm, maxv)
    mv = maxv[...]
    ebase = wid * EPT

    def issue(j, b, first=False):
        if not first:
            @pl.when(j >= 2)
            def _():
                pltpu.make_async_copy(drows[b], acc.at[dstv[b]], semW[b]).wait()
        off = ebase + j * K
        pltpu.sync_copy(src_hbm.at[pl.ds(off, K)], srcv[b])
        pltpu.sync_copy(dst_hbm.at[pl.ds(off, K)], dstv[b])
        pltpu.async_copy(S_hbm.at[srcv[b]], srows[b], semS[b])
        pltpu.async_copy(D_hbm.at[dstv[b]], drows[b], semD[b])

    def run_chunk(j, b):
        sr = srows[b]
        dr = drows[b]
        dv_ref = dstv[b]
        m2 = mrows2[b]

        @pl.when(j >= 2)
        def _():
            pltpu.make_async_copy(m2, acc.at[dstv2[b]], semW2[b]).wait()
        for r in range((K + 15) // 16):
            bb = min(16 * r, K - 16)
            dv = dv_ref[pl.ds(bb, 16)]
            dstv2[b][pl.ds(bb, 16)] = (dv >> 5) + NMSG


        @plsc.parallel_loop(0, K, step=1, unroll=4)
        def edge(i):
            sa = sr[i, pl.ds(HID, 16)]
            dc = dr[i, pl.ds(0, 16)]
            x = sa + dc
            e = jnp.maximum(x, 0.2 * x)
            y = mv + dc
            cc = jnp.maximum(y, 0.2 * y)
            ex = jnp.exp(e - cc)
            base = jnp.minimum((i // 16) * 16, K - 16)
            dvec = dv_ref[pl.ds(base, 16)]
            d_b = jnp.take_along_axis(dvec, jnp.full((16,), i - base, jnp.int32),
                                      axis=0)
            for v in range(8):
                m2[i, pl.ds(v * 16, 16)] = zeros16
            plsc.store_scatter(m2, [jnp.full((16,), i, jnp.int32),
                                    (d_b & 31) * 4 + iota16], ex, mask=dmask)
            for h in range(HEADS):
                bh = _lane_bcast(ex, h)
                dr[i, pl.ds(h * 32, 16)] = sr[i, pl.ds(h * 32, 16)] * bh
                dr[i, pl.ds(h * 32 + 16, 16)] = sr[i, pl.ds(h * 32 + 16, 16)] * bh
        pltpu.async_copy(dr, acc.at[dv_ref], semW[b], add=True)
        pltpu.async_copy(m2, acc.at[dstv2[b]], semW2[b], add=True)

    issue(0, 0, first=True)

    def outer(g, carry):
        for b in range(2):
            j = 2 * g + b
            pltpu.make_async_copy(S_hbm.at[srcv[b]], srows[b], semS[b]).wait()
            pltpu.make_async_copy(D_hbm.at[dstv[b]], drows[b], semD[b]).wait()
            if b == 0:
                issue(j + 1, 1)
            else:
                @pl.when(g < NCHUNK // 2 - 1)
                def _():
                    issue(j + 1, 0)
            run_chunk(j, b)
        return carry
    lax.fori_loop(0, NCHUNK // 2, outer, 0)

    for b in range(2):
        pltpu.make_async_copy(drows[b], acc.at[dstv[b]], semW[b]).wait()
        pltpu.make_async_copy(mrows2[b], acc.at[dstv2[b]], semW2[b]).wait()

    plsc.subcore_barrier()
    for r in range(RPT // ZR):
        start = sid * RPT + r * ZR
        pltpu.sync_copy(acc.at[pl.ds(start, ZR)], out_hbm.at[cid, pl.ds(start, ZR)])


@functools.lru_cache(maxsize=1)
def _sc_edge_fn():
    return pl.kernel(
        _sc_body,
        out_type=jax.ShapeDtypeStruct((2, NACC, HID), jnp.float32),
        mesh=plsc.VectorSubcoreMesh(core_axis_name="c", subcore_axis_name="s"),
        compiler_params=pltpu.CompilerParams(needs_layout_passes=False),
        scratch_types=[
            pltpu.VMEM((K,), jnp.int32),
            pltpu.VMEM((K,), jnp.int32),
            pltpu.VMEM((K,), jnp.int32),
            pltpu.VMEM((K,), jnp.int32),
            pltpu.VMEM((K,), jnp.int32),
            pltpu.VMEM((K,), jnp.int32),
            pltpu.VMEM((K, SROW), jnp.float32),
            pltpu.VMEM((K, SROW), jnp.float32),
            pltpu.VMEM((K, DROW), jnp.float32),
            pltpu.VMEM((K, DROW), jnp.float32),
            pltpu.VMEM((K, HID), jnp.float32),
            pltpu.VMEM((K, HID), jnp.float32),
            pltpu.VMEM((16,), jnp.float32),
            pltpu.VMEM((ZR, HID), jnp.float32),
            pltpu.VMEM_SHARED((NACC, HID), jnp.float32),
            pltpu.SemaphoreType.DMA,
            pltpu.SemaphoreType.DMA,
            pltpu.SemaphoreType.DMA,
            pltpu.SemaphoreType.DMA,
            pltpu.SemaphoreType.DMA,
            pltpu.SemaphoreType.DMA,
            pltpu.SemaphoreType.DMA,
            pltpu.SemaphoreType.DMA,
        ],
    )


def _sc_edge(S, D, mxv, src, dst):
    return _sc_edge_fn()(S, D, mxv, src, dst)


def _mix_mat(a):
    return (a[:, :, None] * jnp.eye(HEADS, 16, dtype=jnp.float32)[:, None, :]).reshape(HID, 16)


def _mx_vec(mx):
    return jnp.concatenate([mx[0, :HEADS], jnp.full((16 - HEADS,), 1e9, jnp.float32)])


def _split_acc(acc):
    msg = acc[:, :N, :]
    nden_rows = (N + 31) // 32
    den = acc[:, NMSG:NMSG + nden_rows, :].reshape(2, nden_rows * HID)
    den = den[:, :N * HEADS].reshape(2, N, HEADS)
    den = jnp.concatenate([den, jnp.zeros((2, N, 4), jnp.float32)], axis=-1)
    return msg, den


def kernel(external_cell_embeds, entity_embeddings, edge_index, row_indices, col_indices,
           entity_W, entity_b, in_W, in_b, ln_g, ln_b,
           W1, b1, as1, ad1, W2, b2, as2, ad2):
    edge = edge_index.astype(jnp.int32)
    src = edge[0]
    dst = edge[1]
    pos = jnp.stack([row_indices.astype(jnp.float32),
                     col_indices.astype(jnp.float32)], axis=-1)

    h0, S1, D1, mx1 = _frontend(
        external_cell_embeds, entity_embeddings, pos,
        entity_W, entity_b.reshape(1, -1), in_W, in_b.reshape(1, -1),
        ln_g.reshape(1, -1), ln_b.reshape(1, -1),
        W1, b1.reshape(1, -1), _mix_mat(as1), _mix_mat(ad1))

    b2r = b2.reshape(1, -1)
    As2, Ad2 = _mix_mat(as2), _mix_mat(ad2)

    nlayers = 2 + jnp.minimum(src[0], 0) * 0

    def cond(carry):
        return carry[0] < nlayers

    def body(carry):
        t, S, D, mxv, h = carry
        acc = _sc_edge(S, D, mxv, src, dst)
        msg, den = _split_acc(acc)
        hn, Sn, Dn, mxn = _combine_prep(msg[0], msg[1], den[0], den[1], h,
                                        W2, b2r, As2, Ad2)
        return (t + 1, Sn, Dn, _mx_vec(mxn), hn)

    carry = lax.while_loop(cond, body, (jnp.int32(0), S1, D1, _mx_vec(mx1), h0))
    return carry[4]
